# Initial kernel scaffold; baseline (speedup 1.0000x reference)
import jax
import jax.numpy as jnp
from jax.experimental import pallas as pl


def kernel(input_ids, emb_table, params, linear_W, linear_b):
    raise NotImplementedError("write your pallas kernel here")


# speedup vs baseline: 16.8796x; 16.8796x over previous
import jax
import jax.numpy as jnp
from jax.experimental import pallas as pl
from jax.experimental.pallas import tpu as pltpu

B = 16
S = 2048
H = 128
G = 4 * H
V = 128
CT = 256
NB = S // CT


def _gates(z, c):
    i = jax.nn.sigmoid(z[:, 0 * H:1 * H])
    f = jax.nn.sigmoid(z[:, 1 * H:2 * H])
    g = jnp.tanh(z[:, 2 * H:3 * H])
    o = jax.nn.sigmoid(z[:, 3 * H:4 * H])
    c_new = f * c + i * g
    h_new = o * jnp.tanh(c_new)
    return h_new, c_new


def _l0_kernel(idf_ref, idb_ref, tf_ref, tb_ref, wf_ref, wb_ref,
               hf_out, hb_out, pf_ref, pb_ref, st_ref):
    i = pl.program_id(0)

    iota = jax.lax.broadcasted_iota(jnp.int32, (CT, B, V), 2)
    ohf = (idf_ref[0][:, :, None] == iota).astype(jnp.float32).reshape(CT * B, V)
    pf_ref[...] = jnp.dot(ohf, tf_ref[...],
                          preferred_element_type=jnp.float32).reshape(CT, B, G)
    ohb = (idb_ref[0][:, :, None] == iota).astype(jnp.float32).reshape(CT * B, V)
    pb_ref[...] = jnp.dot(ohb, tb_ref[...],
                          preferred_element_type=jnp.float32).reshape(CT, B, G)

    @pl.when(i == 0)
    def _():
        st_ref[...] = jnp.zeros((4, B, H), jnp.float32)

    wfT = wf_ref[...]
    wbT = wb_ref[...]

    def step(j, carry):
        hf, cf, hb, cb = carry
        jb = CT - 1 - j
        zf = pf_ref[j] + jnp.dot(hf, wfT, preferred_element_type=jnp.float32)
        zb = pb_ref[jb] + jnp.dot(hb, wbT, preferred_element_type=jnp.float32)
        hf, cf = _gates(zf, cf)
        hb, cb = _gates(zb, cb)
        hf_out[0, j] = hf
        hb_out[0, jb] = hb
        return hf, cf, hb, cb

    init = (st_ref[0], st_ref[1], st_ref[2], st_ref[3])
    hf, cf, hb, cb = jax.lax.fori_loop(0, CT, step, init)
    st_ref[0] = hf
    st_ref[1] = cf
    st_ref[2] = hb
    st_ref[3] = cb


def _l1_kernel(hfi_ref, hbi_ref, hfr_ref, hbr_ref,
               af_ref, bf_ref, ab_ref, bb_ref,
               wf_ref, wb_ref, biasf_ref, biasb_ref,
               wlf_ref, wlb_ref, lb_ref,
               out_ref, pf_ref, pb_ref, h2f_ref, h2b_ref, st_ref):
    i = pl.program_id(0)

    xf1 = hfi_ref[0].reshape(CT * B, H)
    xf2 = hbi_ref[0].reshape(CT * B, H)
    pf = (jnp.dot(xf1, af_ref[...], preferred_element_type=jnp.float32)
          + jnp.dot(xf2, bf_ref[...], preferred_element_type=jnp.float32)
          + biasf_ref[...])
    pf_ref[...] = pf.reshape(CT, B, G)

    xb1 = hfr_ref[0].reshape(CT * B, H)
    xb2 = hbr_ref[0].reshape(CT * B, H)
    pb = (jnp.dot(xb1, ab_ref[...], preferred_element_type=jnp.float32)
          + jnp.dot(xb2, bb_ref[...], preferred_element_type=jnp.float32)
          + biasb_ref[...])
    pb_ref[...] = pb.reshape(CT, B, G)

    @pl.when(i == 0)
    def _():
        st_ref[...] = jnp.zeros((4, B, H), jnp.float32)
        out_ref[...] = jnp.zeros((NB, CT, B), jnp.float32)

    wfT = wf_ref[...]
    wbT = wb_ref[...]

    def step(j, carry):
        hf, cf, hb, cb = carry
        jb = CT - 1 - j
        zf = pf_ref[j] + jnp.dot(hf, wfT, preferred_element_type=jnp.float32)
        zb = pb_ref[jb] + jnp.dot(hb, wbT, preferred_element_type=jnp.float32)
        hf, cf = _gates(zf, cf)
        hb, cb = _gates(zb, cb)
        h2f_ref[j] = hf
        h2b_ref[jb] = hb
        return hf, cf, hb, cb

    init = (st_ref[0], st_ref[1], st_ref[2], st_ref[3])
    hf, cf, hb, cb = jax.lax.fori_loop(0, CT, step, init)
    st_ref[0] = hf
    st_ref[1] = cf
    st_ref[2] = hb
    st_ref[3] = cb

    lb = lb_ref[0, 0]
    of_blk = jnp.sum(h2f_ref[...] * wlf_ref[...][None], axis=2) + lb
    ob_blk = jnp.sum(h2b_ref[...] * wlb_ref[...][None], axis=2)
    out_ref[i] = out_ref[i] + of_blk
    out_ref[NB - 1 - i] = out_ref[NB - 1 - i] + ob_blk


def kernel(input_ids, emb_table, params, linear_W, linear_b):
    f32 = jnp.float32
    ids3 = input_ids.astype(jnp.int32).T.reshape(NB, CT, B)

    (wih_f0, whh_f0, bih_f0, bhh_f0), (wih_b0, whh_b0, bih_b0, bhh_b0) = params[0]
    (wih_f1, whh_f1, bih_f1, bhh_f1), (wih_b1, whh_b1, bih_b1, bhh_b1) = params[1]

    tf = emb_table @ wih_f0.T + (bih_f0 + bhh_f0)[None, :]
    tb = emb_table @ wih_b0.T + (bih_b0 + bhh_b0)[None, :]

    wspec = pl.BlockSpec((H, G), lambda i: (0, 0))
    idspec_f = pl.BlockSpec((1, CT, B), lambda i: (i, 0, 0))
    idspec_b = pl.BlockSpec((1, CT, B), lambda i: (NB - 1 - i, 0, 0))
    hspec_f = pl.BlockSpec((1, CT, B, H), lambda i: (i, 0, 0, 0))
    hspec_b = pl.BlockSpec((1, CT, B, H), lambda i: (NB - 1 - i, 0, 0, 0))

    hf, hb = pl.pallas_call(
        _l0_kernel,
        grid=(NB,),
        in_specs=[idspec_f, idspec_b, wspec, wspec, wspec, wspec],
        out_specs=[hspec_f, hspec_b],
        out_shape=[jax.ShapeDtypeStruct((NB, CT, B, H), f32)] * 2,
        scratch_shapes=[
            pltpu.VMEM((CT, B, G), f32),
            pltpu.VMEM((CT, B, G), f32),
            pltpu.VMEM((4, B, H), f32),
        ],
        compiler_params=pltpu.CompilerParams(
            dimension_semantics=("arbitrary",)),
    )(ids3, ids3, tf, tb, whh_f0.T, whh_b0.T)

    bspec = pl.BlockSpec((1, G), lambda i: (0, 0))
    wlspec = pl.BlockSpec((1, H), lambda i: (0, 0))

    out3 = pl.pallas_call(
        _l1_kernel,
        grid=(NB,),
        in_specs=[hspec_f, hspec_f, hspec_b, hspec_b,
                  wspec, wspec, wspec, wspec, wspec, wspec,
                  bspec, bspec, wlspec, wlspec,
                  pl.BlockSpec((1, 1), lambda i: (0, 0))],
        out_specs=pl.BlockSpec((NB, CT, B), lambda i: (0, 0, 0)),
        out_shape=jax.ShapeDtypeStruct((NB, CT, B), f32),
        scratch_shapes=[
            pltpu.VMEM((CT, B, G), f32),
            pltpu.VMEM((CT, B, G), f32),
            pltpu.VMEM((CT, B, H), f32),
            pltpu.VMEM((CT, B, H), f32),
            pltpu.VMEM((4, B, H), f32),
        ],
        compiler_params=pltpu.CompilerParams(
            dimension_semantics=("arbitrary",)),
    )(hf, hb, hf, hb,
      wih_f1[:, :H].T, wih_f1[:, H:].T, wih_b1[:, :H].T, wih_b1[:, H:].T,
      whh_f1.T, whh_b1.T,
      (bih_f1 + bhh_f1)[None, :], (bih_b1 + bhh_b1)[None, :],
      linear_W[:, :H], linear_W[:, H:], linear_b.reshape(1, 1))

    return out3.reshape(S, B).T


# speedup vs baseline: 20.6178x; 1.2215x over previous
import jax
import jax.numpy as jnp
from jax.experimental import pallas as pl
from jax.experimental.pallas import tpu as pltpu

B = 16
S = 2048
H = 128
G = 4 * H
V = 128
CT = 32
NB = S // CT

BF = jnp.bfloat16
F32 = jnp.float32


def _pack_rec_rhs(whh):
    wt = whh.T
    return jnp.concatenate([wt[:, :2 * H], wt[:, 2 * H:]], axis=0).astype(BF)


def _pack_proj_rhs(t):
    z = jnp.zeros((128, 2 * H), F32)
    a = jnp.concatenate([t[:, :2 * H], z], axis=0).astype(BF)
    b = jnp.concatenate([t[:, 2 * H:], z], axis=0).astype(BF)
    return a, b


def _proj_into_mrb(lhs_f, lhs_b, pa_f, pb_f, pa_b, pb_b):
    pltpu.matmul_push_rhs(pa_f, staging_register=0, mxu_index=0)
    pltpu.matmul_push_rhs(pb_f, staging_register=1, mxu_index=0)
    pltpu.matmul_push_rhs(pa_b, staging_register=0, mxu_index=1)
    pltpu.matmul_push_rhs(pb_b, staging_register=1, mxu_index=1)
    pltpu.matmul_acc_lhs(0, lhs_f, 0, load_staged_rhs=0)
    pltpu.matmul_acc_lhs(128, lhs_f, 0, load_staged_rhs=1)
    pltpu.matmul_acc_lhs(0, lhs_b, 1, load_staged_rhs=0)
    pltpu.matmul_acc_lhs(128, lhs_b, 1, load_staged_rhs=1)


def _scan_chunk(wfp_ref, wbp_ref, st_ref, init_state, emit, bias_f=None,
                bias_b=None):
    pltpu.matmul_push_rhs(wfp_ref[...], staging_register=0, mxu_index=0)
    pltpu.matmul_push_rhs(wbp_ref[...], staging_register=0, mxu_index=1)

    hf, cf, hb, cb = init_state
    zeros = jnp.zeros((B, H), BF)
    for j in range(CT):
        jb = CT - 1 - j
        first = 0 if j == 0 else None
        hf16 = hf.astype(BF)
        hb16 = hb.astype(BF)
        pltpu.matmul_acc_lhs(4 * j, jnp.concatenate([hf16, zeros], 1), 0,
                             load_staged_rhs=first)
        pltpu.matmul_acc_lhs(128 + 4 * j, jnp.concatenate([zeros, hf16], 1), 0)
        pltpu.matmul_acc_lhs(4 * jb, jnp.concatenate([hb16, zeros], 1), 1,
                             load_staged_rhs=first)
        pltpu.matmul_acc_lhs(128 + 4 * jb, jnp.concatenate([zeros, hb16], 1), 1)
        zf0 = pltpu.matmul_pop(4 * j, (B, 2 * H), F32, 0)
        zf1 = pltpu.matmul_pop(128 + 4 * j, (B, 2 * H), F32, 0)
        zb0 = pltpu.matmul_pop(4 * jb, (B, 2 * H), F32, 1)
        zb1 = pltpu.matmul_pop(128 + 4 * jb, (B, 2 * H), F32, 1)
        zf = jnp.concatenate([zf0, zf1], 1)
        zb = jnp.concatenate([zb0, zb1], 1)
        if bias_f is not None:
            zf = zf + bias_f
            zb = zb + bias_b
        hf, cf = _gates(zf, cf)
        hb, cb = _gates(zb, cb)
        emit(j, jb, hf, hb)

    st_ref[0] = hf
    st_ref[1] = cf
    st_ref[2] = hb
    st_ref[3] = cb


def _gates(z, c):
    i = jax.nn.sigmoid(z[:, 0 * H:1 * H])
    f = jax.nn.sigmoid(z[:, 1 * H:2 * H])
    g = jnp.tanh(z[:, 2 * H:3 * H])
    o = jax.nn.sigmoid(z[:, 3 * H:4 * H])
    c_new = f * c + i * g
    h_new = o * jnp.tanh(c_new)
    return h_new, c_new


def _l0_kernel(idf_ref, idb_ref, tfa_ref, tfb_ref, tba_ref, tbb_ref,
               wfp_ref, wbp_ref, hf_out, hb_out, st_ref):
    i = pl.program_id(0)

    @pl.when(i == 0)
    def _():
        st_ref[...] = jnp.zeros((4, B, H), F32)

    iota = jax.lax.broadcasted_iota(jnp.int32, (CT, B, V), 2)
    ohf = (idf_ref[0][:, :, None] == iota).astype(BF).reshape(CT * B, V)
    ohb = (idb_ref[0][:, :, None] == iota).astype(BF).reshape(CT * B, V)
    zpad = jnp.zeros((CT * B, V), BF)
    lhs_f = jnp.concatenate([ohf, zpad], axis=1)
    lhs_b = jnp.concatenate([ohb, zpad], axis=1)
    _proj_into_mrb(lhs_f, lhs_b, tfa_ref[...], tfb_ref[...],
                   tba_ref[...], tbb_ref[...])

    def emit(j, jb, hf, hb):
        hf_out[0, j] = hf
        hb_out[0, jb] = hb

    init = (st_ref[0], st_ref[1], st_ref[2], st_ref[3])
    _scan_chunk(wfp_ref, wbp_ref, st_ref, init, emit)


def _l1_kernel(hfi_ref, hbi_ref, hfr_ref, hbr_ref,
               wfa_ref, wfb_ref, wba_ref, wbb_ref,
               wfp_ref, wbp_ref,
               biasf_ref, biasb_ref,
               wlf_ref, wlb_ref, lb_ref,
               out_ref, h2f_ref, h2b_ref, st_ref):
    i = pl.program_id(0)

    @pl.when(i == 0)
    def _():
        st_ref[...] = jnp.zeros((4, B, H), F32)
        out_ref[...] = jnp.zeros((NB, CT, B), F32)

    lhs_f = jnp.concatenate([hfi_ref[0], hbi_ref[0]],
                            axis=2).astype(BF).reshape(CT * B, 2 * H)
    lhs_b = jnp.concatenate([hfr_ref[0], hbr_ref[0]],
                            axis=2).astype(BF).reshape(CT * B, 2 * H)
    _proj_into_mrb(lhs_f, lhs_b, wfa_ref[...], wfb_ref[...],
                   wba_ref[...], wbb_ref[...])

    def emit(j, jb, hf, hb):
        h2f_ref[j] = hf
        h2b_ref[jb] = hb

    init = (st_ref[0], st_ref[1], st_ref[2], st_ref[3])
    _scan_chunk(wfp_ref, wbp_ref, st_ref, init, emit,
                bias_f=biasf_ref[...], bias_b=biasb_ref[...])

    lb = lb_ref[0, 0]
    of_blk = jnp.sum(h2f_ref[...] * wlf_ref[...][None], axis=2) + lb
    ob_blk = jnp.sum(h2b_ref[...] * wlb_ref[...][None], axis=2)
    out_ref[i] = out_ref[i] + of_blk
    out_ref[NB - 1 - i] = out_ref[NB - 1 - i] + ob_blk


def kernel(input_ids, emb_table, params, linear_W, linear_b):
    ids3 = input_ids.astype(jnp.int32).T.reshape(NB, CT, B)

    (wih_f0, whh_f0, bih_f0, bhh_f0), (wih_b0, whh_b0, bih_b0, bhh_b0) = params[0]
    (wih_f1, whh_f1, bih_f1, bhh_f1), (wih_b1, whh_b1, bih_b1, bhh_b1) = params[1]

    tf = emb_table @ wih_f0.T + (bih_f0 + bhh_f0)[None, :]
    tb = emb_table @ wih_b0.T + (bih_b0 + bhh_b0)[None, :]
    tfa, tfb = _pack_proj_rhs(tf)
    tba, tbb = _pack_proj_rhs(tb)

    w1fa = wih_f1.T[:, :2 * H].astype(BF)
    w1fb = wih_f1.T[:, 2 * H:].astype(BF)
    w1ba = wih_b1.T[:, :2 * H].astype(BF)
    w1bb = wih_b1.T[:, 2 * H:].astype(BF)

    wf0p = _pack_rec_rhs(whh_f0)
    wb0p = _pack_rec_rhs(whh_b0)
    wf1p = _pack_rec_rhs(whh_f1)
    wb1p = _pack_rec_rhs(whh_b1)

    wpspec = pl.BlockSpec((2 * H, 2 * H), lambda i: (0, 0))
    idspec_f = pl.BlockSpec((1, CT, B), lambda i: (i, 0, 0))
    idspec_b = pl.BlockSpec((1, CT, B), lambda i: (NB - 1 - i, 0, 0))
    hspec_f = pl.BlockSpec((1, CT, B, H), lambda i: (i, 0, 0, 0))
    hspec_b = pl.BlockSpec((1, CT, B, H), lambda i: (NB - 1 - i, 0, 0, 0))

    hf, hb = pl.pallas_call(
        _l0_kernel,
        grid=(NB,),
        in_specs=[idspec_f, idspec_b,
                  wpspec, wpspec, wpspec, wpspec, wpspec, wpspec],
        out_specs=[hspec_f, hspec_b],
        out_shape=[jax.ShapeDtypeStruct((NB, CT, B, H), F32)] * 2,
        scratch_shapes=[pltpu.VMEM((4, B, H), F32)],
        compiler_params=pltpu.CompilerParams(
            dimension_semantics=("arbitrary",)),
    )(ids3, ids3, tfa, tfb, tba, tbb, wf0p, wb0p)

    bspec = pl.BlockSpec((1, G), lambda i: (0, 0))
    wlspec = pl.BlockSpec((1, H), lambda i: (0, 0))

    out3 = pl.pallas_call(
        _l1_kernel,
        grid=(NB,),
        in_specs=[hspec_f, hspec_f, hspec_b, hspec_b,
                  wpspec, wpspec, wpspec, wpspec, wpspec, wpspec,
                  bspec, bspec, wlspec, wlspec,
                  pl.BlockSpec((1, 1), lambda i: (0, 0))],
        out_specs=pl.BlockSpec((NB, CT, B), lambda i: (0, 0, 0)),
        out_shape=jax.ShapeDtypeStruct((NB, CT, B), F32),
        scratch_shapes=[
            pltpu.VMEM((CT, B, H), F32),
            pltpu.VMEM((CT, B, H), F32),
            pltpu.VMEM((4, B, H), F32),
        ],
        compiler_params=pltpu.CompilerParams(
            dimension_semantics=("arbitrary",)),
    )(hf, hb, hf, hb,
      w1fa, w1fb, w1ba, w1bb,
      wf1p, wb1p,
      (bih_f1 + bhh_f1)[None, :], (bih_b1 + bhh_b1)[None, :],
      linear_W[:, :H], linear_W[:, H:], linear_b.reshape(1, 1))

    return out3.reshape(S, B).T


# speedup vs baseline: 21.2835x; 1.0323x over previous
import jax
import jax.numpy as jnp
from jax.experimental import pallas as pl
from jax.experimental.pallas import tpu as pltpu

B = 16
S = 2048
H = 128
G = 4 * H
V = 128
CT = 32
NB = S // CT

BF = jnp.bfloat16
F32 = jnp.float32


def _pack_rec_rhs(whh):
    wt = whh.T
    return jnp.concatenate([wt[:, :2 * H], wt[:, 2 * H:]], axis=0).astype(BF)


def _pack_proj_rhs(t):
    z = jnp.zeros((128, 2 * H), F32)
    a = jnp.concatenate([t[:, :2 * H], z], axis=0).astype(BF)
    b = jnp.concatenate([t[:, 2 * H:], z], axis=0).astype(BF)
    return a, b


def _proj_into_mrb(lhs_f, lhs_b, pa_f, pb_f, pa_b, pb_b):
    pltpu.matmul_push_rhs(pa_f, staging_register=0, mxu_index=0)
    pltpu.matmul_push_rhs(pb_f, staging_register=1, mxu_index=0)
    pltpu.matmul_push_rhs(pa_b, staging_register=0, mxu_index=1)
    pltpu.matmul_push_rhs(pb_b, staging_register=1, mxu_index=1)
    pltpu.matmul_acc_lhs(0, lhs_f, 0, load_staged_rhs=0)
    pltpu.matmul_acc_lhs(128, lhs_f, 0, load_staged_rhs=1)
    pltpu.matmul_acc_lhs(0, lhs_b, 1, load_staged_rhs=0)
    pltpu.matmul_acc_lhs(128, lhs_b, 1, load_staged_rhs=1)


def _scan_chunk(wfp_ref, wbp_ref, st_ref, init_state, emit, bias_f=None,
                bias_b=None):
    pltpu.matmul_push_rhs(wfp_ref[...], staging_register=0, mxu_index=0)
    pltpu.matmul_push_rhs(wbp_ref[...], staging_register=0, mxu_index=1)

    hf, cf, hb, cb = init_state
    zeros = jnp.zeros((B, H), BF)
    for j in range(CT):
        jb = CT - 1 - j
        first = 0 if j == 0 else None
        hf16 = hf.astype(BF)
        hb16 = hb.astype(BF)
        pltpu.matmul_acc_lhs(4 * j, jnp.concatenate([hf16, zeros], 1), 0,
                             load_staged_rhs=first)
        pltpu.matmul_acc_lhs(128 + 4 * j, jnp.concatenate([zeros, hf16], 1), 0)
        pltpu.matmul_acc_lhs(4 * jb, jnp.concatenate([hb16, zeros], 1), 1,
                             load_staged_rhs=first)
        pltpu.matmul_acc_lhs(128 + 4 * jb, jnp.concatenate([zeros, hb16], 1), 1)
        zf0 = pltpu.matmul_pop(4 * j, (B, 2 * H), F32, 0)
        zf1 = pltpu.matmul_pop(128 + 4 * j, (B, 2 * H), F32, 0)
        zb0 = pltpu.matmul_pop(4 * jb, (B, 2 * H), F32, 1)
        zb1 = pltpu.matmul_pop(128 + 4 * jb, (B, 2 * H), F32, 1)
        if bias_f is not None:
            zf0 = zf0 + bias_f[:, :2 * H]
            zf1 = zf1 + bias_f[:, 2 * H:]
            zb0 = zb0 + bias_b[:, :2 * H]
            zb1 = zb1 + bias_b[:, 2 * H:]
        hf, cf = _gates(zf0, zf1, cf)
        hb, cb = _gates(zb0, zb1, cb)
        emit(j, jb, hf, hb)

    st_ref[0] = hf
    st_ref[1] = cf
    st_ref[2] = hb
    st_ref[3] = cb


def _sig(x):
    return 0.5 * jnp.tanh(0.5 * x) + 0.5


def _gates(z0, z1, c):
    i = _sig(z0[:, :H])
    f = _sig(z0[:, H:])
    g = jnp.tanh(z1[:, :H])
    o = _sig(z1[:, H:])
    c_new = f * c + i * g
    h_new = o * jnp.tanh(c_new)
    return h_new, c_new


def _l0_kernel(idf_ref, idb_ref, tfa_ref, tfb_ref, tba_ref, tbb_ref,
               wfp_ref, wbp_ref, hf_out, hb_out, st_ref):
    i = pl.program_id(0)

    @pl.when(i == 0)
    def _():
        st_ref[...] = jnp.zeros((4, B, H), F32)

    iota = jax.lax.broadcasted_iota(jnp.int32, (CT, B, V), 2)
    ohf = (idf_ref[0][:, :, None] == iota).astype(BF).reshape(CT * B, V)
    ohb = (idb_ref[0][:, :, None] == iota).astype(BF).reshape(CT * B, V)
    zpad = jnp.zeros((CT * B, V), BF)
    lhs_f = jnp.concatenate([ohf, zpad], axis=1)
    lhs_b = jnp.concatenate([ohb, zpad], axis=1)
    _proj_into_mrb(lhs_f, lhs_b, tfa_ref[...], tfb_ref[...],
                   tba_ref[...], tbb_ref[...])

    def emit(j, jb, hf, hb):
        hf_out[0, j] = hf
        hb_out[0, jb] = hb

    init = (st_ref[0], st_ref[1], st_ref[2], st_ref[3])
    _scan_chunk(wfp_ref, wbp_ref, st_ref, init, emit)


def _l1_kernel(hfi_ref, hbi_ref, hfr_ref, hbr_ref,
               wfa_ref, wfb_ref, wba_ref, wbb_ref,
               wfp_ref, wbp_ref,
               biasf_ref, biasb_ref,
               wlf_ref, wlb_ref, lb_ref,
               out_ref, h2f_ref, h2b_ref, st_ref):
    i = pl.program_id(0)

    @pl.when(i == 0)
    def _():
        st_ref[...] = jnp.zeros((4, B, H), F32)
        out_ref[...] = jnp.zeros((NB, CT, B), F32)

    lhs_f = jnp.concatenate([hfi_ref[0], hbi_ref[0]],
                            axis=2).astype(BF).reshape(CT * B, 2 * H)
    lhs_b = jnp.concatenate([hfr_ref[0], hbr_ref[0]],
                            axis=2).astype(BF).reshape(CT * B, 2 * H)
    _proj_into_mrb(lhs_f, lhs_b, wfa_ref[...], wfb_ref[...],
                   wba_ref[...], wbb_ref[...])

    def emit(j, jb, hf, hb):
        h2f_ref[j] = hf
        h2b_ref[jb] = hb

    init = (st_ref[0], st_ref[1], st_ref[2], st_ref[3])
    _scan_chunk(wfp_ref, wbp_ref, st_ref, init, emit,
                bias_f=biasf_ref[...], bias_b=biasb_ref[...])

    lb = lb_ref[0, 0]
    of_blk = jnp.sum(h2f_ref[...] * wlf_ref[...][None], axis=2) + lb
    ob_blk = jnp.sum(h2b_ref[...] * wlb_ref[...][None], axis=2)
    out_ref[i] = out_ref[i] + of_blk
    out_ref[NB - 1 - i] = out_ref[NB - 1 - i] + ob_blk


def kernel(input_ids, emb_table, params, linear_W, linear_b):
    ids3 = input_ids.astype(jnp.int32).T.reshape(NB, CT, B)

    (wih_f0, whh_f0, bih_f0, bhh_f0), (wih_b0, whh_b0, bih_b0, bhh_b0) = params[0]
    (wih_f1, whh_f1, bih_f1, bhh_f1), (wih_b1, whh_b1, bih_b1, bhh_b1) = params[1]

    tf = emb_table @ wih_f0.T + (bih_f0 + bhh_f0)[None, :]
    tb = emb_table @ wih_b0.T + (bih_b0 + bhh_b0)[None, :]
    tfa, tfb = _pack_proj_rhs(tf)
    tba, tbb = _pack_proj_rhs(tb)

    w1fa = wih_f1.T[:, :2 * H].astype(BF)
    w1fb = wih_f1.T[:, 2 * H:].astype(BF)
    w1ba = wih_b1.T[:, :2 * H].astype(BF)
    w1bb = wih_b1.T[:, 2 * H:].astype(BF)

    wf0p = _pack_rec_rhs(whh_f0)
    wb0p = _pack_rec_rhs(whh_b0)
    wf1p = _pack_rec_rhs(whh_f1)
    wb1p = _pack_rec_rhs(whh_b1)

    wpspec = pl.BlockSpec((2 * H, 2 * H), lambda i: (0, 0))
    idspec_f = pl.BlockSpec((1, CT, B), lambda i: (i, 0, 0))
    idspec_b = pl.BlockSpec((1, CT, B), lambda i: (NB - 1 - i, 0, 0))
    hspec_f = pl.BlockSpec((1, CT, B, H), lambda i: (i, 0, 0, 0))
    hspec_b = pl.BlockSpec((1, CT, B, H), lambda i: (NB - 1 - i, 0, 0, 0))

    hf, hb = pl.pallas_call(
        _l0_kernel,
        grid=(NB,),
        in_specs=[idspec_f, idspec_b,
                  wpspec, wpspec, wpspec, wpspec, wpspec, wpspec],
        out_specs=[hspec_f, hspec_b],
        out_shape=[jax.ShapeDtypeStruct((NB, CT, B, H), F32)] * 2,
        scratch_shapes=[pltpu.VMEM((4, B, H), F32)],
        compiler_params=pltpu.CompilerParams(
            dimension_semantics=("arbitrary",)),
    )(ids3, ids3, tfa, tfb, tba, tbb, wf0p, wb0p)

    bspec = pl.BlockSpec((1, G), lambda i: (0, 0))
    wlspec = pl.BlockSpec((1, H), lambda i: (0, 0))

    out3 = pl.pallas_call(
        _l1_kernel,
        grid=(NB,),
        in_specs=[hspec_f, hspec_f, hspec_b, hspec_b,
                  wpspec, wpspec, wpspec, wpspec, wpspec, wpspec,
                  bspec, bspec, wlspec, wlspec,
                  pl.BlockSpec((1, 1), lambda i: (0, 0))],
        out_specs=pl.BlockSpec((NB, CT, B), lambda i: (0, 0, 0)),
        out_shape=jax.ShapeDtypeStruct((NB, CT, B), F32),
        scratch_shapes=[
            pltpu.VMEM((CT, B, H), F32),
            pltpu.VMEM((CT, B, H), F32),
            pltpu.VMEM((4, B, H), F32),
        ],
        compiler_params=pltpu.CompilerParams(
            dimension_semantics=("arbitrary",)),
    )(hf, hb, hf, hb,
      w1fa, w1fb, w1ba, w1bb,
      wf1p, wb1p,
      (bih_f1 + bhh_f1)[None, :], (bih_b1 + bhh_b1)[None, :],
      linear_W[:, :H], linear_W[:, H:], linear_b.reshape(1, 1))

    return out3.reshape(S, B).T


# speedup vs baseline: 21.5606x; 1.0130x over previous
import jax
import jax.numpy as jnp
from jax.experimental import pallas as pl
from jax.experimental.pallas import tpu as pltpu

B = 16
S = 2048
H = 128
G = 4 * H
V = 128
CT = 32
NB = S // CT

BF = jnp.bfloat16
F32 = jnp.float32


def _pack_stack(wt):
    return jnp.concatenate([wt[:, :2 * H], wt[:, 2 * H:]], axis=0).astype(BF)


def _proj_into_mrb(lhs0_f, lhs1_f, lhs0_b, lhs1_b, pa_f, pb_f, pa_b, pb_b,
                   split_cols):
    pltpu.matmul_push_rhs(pa_f, staging_register=0, mxu_index=0)
    pltpu.matmul_push_rhs(pa_b, staging_register=0, mxu_index=1)
    if split_cols:
        pltpu.matmul_acc_lhs(0, lhs0_f, 0, load_staged_rhs=0)
        pltpu.matmul_acc_lhs(128, lhs1_f, 0)
        pltpu.matmul_acc_lhs(0, lhs0_b, 1, load_staged_rhs=0)
        pltpu.matmul_acc_lhs(128, lhs1_b, 1)
    else:
        pltpu.matmul_push_rhs(pb_f, staging_register=1, mxu_index=0)
        pltpu.matmul_push_rhs(pb_b, staging_register=1, mxu_index=1)
        pltpu.matmul_acc_lhs(0, lhs0_f, 0, load_staged_rhs=0)
        pltpu.matmul_acc_lhs(128, lhs1_f, 0, load_staged_rhs=1)
        pltpu.matmul_acc_lhs(0, lhs0_b, 1, load_staged_rhs=0)
        pltpu.matmul_acc_lhs(128, lhs1_b, 1, load_staged_rhs=1)


def _scan_chunk(wfp_ref, wbp_ref, stc_ref, sth_ref, emit, bias_f=None,
                bias_b=None):
    pltpu.matmul_push_rhs(wfp_ref[...], staging_register=0, mxu_index=0)
    pltpu.matmul_push_rhs(wbp_ref[...], staging_register=0, mxu_index=1)

    hf16 = sth_ref[0]
    hb16 = sth_ref[1]
    cf = stc_ref[0]
    cb = stc_ref[1]
    zeros = jnp.zeros((B, H), BF)
    for j in range(CT):
        jb = CT - 1 - j
        first = 0 if j == 0 else None
        pltpu.matmul_acc_lhs(4 * j, jnp.concatenate([hf16, zeros], 1), 0,
                             load_staged_rhs=first)
        pltpu.matmul_acc_lhs(128 + 4 * j, jnp.concatenate([zeros, hf16], 1), 0)
        pltpu.matmul_acc_lhs(4 * jb, jnp.concatenate([hb16, zeros], 1), 1,
                             load_staged_rhs=first)
        pltpu.matmul_acc_lhs(128 + 4 * jb, jnp.concatenate([zeros, hb16], 1), 1)
        zf0 = pltpu.matmul_pop(4 * j, (B, 2 * H), F32, 0)
        zf1 = pltpu.matmul_pop(128 + 4 * j, (B, 2 * H), F32, 0)
        zb0 = pltpu.matmul_pop(4 * jb, (B, 2 * H), F32, 1)
        zb1 = pltpu.matmul_pop(128 + 4 * jb, (B, 2 * H), F32, 1)
        if bias_f is not None:
            zf0 = zf0 + bias_f[:, :2 * H]
            zf1 = zf1 + bias_f[:, 2 * H:]
            zb0 = zb0 + bias_b[:, :2 * H]
            zb1 = zb1 + bias_b[:, 2 * H:]
        hf, cf = _gates(zf0, zf1, cf)
        hb, cb = _gates(zb0, zb1, cb)
        hf16 = hf.astype(BF)
        hb16 = hb.astype(BF)
        emit(j, jb, hf, hb, hf16, hb16)

    sth_ref[0] = hf16
    sth_ref[1] = hb16
    stc_ref[0] = cf
    stc_ref[1] = cb


def _sig(x):
    return 0.5 * jnp.tanh(0.5 * x) + 0.5


def _gates(z0, z1, c):
    i = _sig(z0[:, :H])
    f = _sig(z0[:, H:])
    g = jnp.tanh(z1[:, :H])
    o = _sig(z1[:, H:])
    c_new = f * c + i * g
    h_new = o * jnp.tanh(c_new)
    return h_new, c_new


def _l0_kernel(idf_ref, idb_ref, tfa_ref, tba_ref,
               wfp_ref, wbp_ref, hf_out, hb_out, stc_ref, sth_ref):
    i = pl.program_id(0)

    @pl.when(i == 0)
    def _():
        stc_ref[...] = jnp.zeros((2, B, H), F32)
        sth_ref[...] = jnp.zeros((2, B, H), BF)

    iota = jax.lax.broadcasted_iota(jnp.int32, (CT, B, V), 2)
    ohf = (idf_ref[0][:, :, None] == iota).astype(BF).reshape(CT * B, V)
    ohb = (idb_ref[0][:, :, None] == iota).astype(BF).reshape(CT * B, V)
    zpad = jnp.zeros((CT * B, V), BF)
    lhs0_f = jnp.concatenate([ohf, zpad], axis=1)
    lhs1_f = jnp.concatenate([zpad, ohf], axis=1)
    lhs0_b = jnp.concatenate([ohb, zpad], axis=1)
    lhs1_b = jnp.concatenate([zpad, ohb], axis=1)
    _proj_into_mrb(lhs0_f, lhs1_f, lhs0_b, lhs1_b,
                   tfa_ref[...], None, tba_ref[...], None, True)

    def emit(j, jb, hf, hb, hf16, hb16):
        hf_out[0, j] = hf16
        hb_out[0, jb] = hb16

    _scan_chunk(wfp_ref, wbp_ref, stc_ref, sth_ref, emit)


def _l1_kernel(hfi_ref, hbi_ref, hfr_ref, hbr_ref,
               wfa_ref, wfb_ref, wba_ref, wbb_ref,
               wfp_ref, wbp_ref,
               biasf_ref, biasb_ref,
               wlf_ref, wlb_ref, lb_ref,
               out_ref, h2f_ref, h2b_ref, stc_ref, sth_ref):
    i = pl.program_id(0)

    @pl.when(i == 0)
    def _():
        stc_ref[...] = jnp.zeros((2, B, H), F32)
        sth_ref[...] = jnp.zeros((2, B, H), BF)
        out_ref[...] = jnp.zeros((NB, CT, B), F32)

    lhs_f = jnp.concatenate([hfi_ref[0], hbi_ref[0]],
                            axis=2).reshape(CT * B, 2 * H)
    lhs_b = jnp.concatenate([hfr_ref[0], hbr_ref[0]],
                            axis=2).reshape(CT * B, 2 * H)
    _proj_into_mrb(lhs_f, lhs_f, lhs_b, lhs_b, wfa_ref[...], wfb_ref[...],
                   wba_ref[...], wbb_ref[...], False)

    def emit(j, jb, hf, hb, hf16, hb16):
        h2f_ref[j] = hf
        h2b_ref[jb] = hb

    _scan_chunk(wfp_ref, wbp_ref, stc_ref, sth_ref, emit,
                bias_f=biasf_ref[...], bias_b=biasb_ref[...])

    lb = lb_ref[0, 0]
    of_blk = jnp.sum(h2f_ref[...] * wlf_ref[...][None], axis=2) + lb
    ob_blk = jnp.sum(h2b_ref[...] * wlb_ref[...][None], axis=2)
    out_ref[i] = out_ref[i] + of_blk
    out_ref[NB - 1 - i] = out_ref[NB - 1 - i] + ob_blk


def kernel(input_ids, emb_table, params, linear_W, linear_b):
    ids3 = input_ids.astype(jnp.int32).T.reshape(NB, CT, B)

    (wih_f0, whh_f0, bih_f0, bhh_f0), (wih_b0, whh_b0, bih_b0, bhh_b0) = params[0]
    (wih_f1, whh_f1, bih_f1, bhh_f1), (wih_b1, whh_b1, bih_b1, bhh_b1) = params[1]

    tf = emb_table @ wih_f0.T + (bih_f0 + bhh_f0)[None, :]
    tb = emb_table @ wih_b0.T + (bih_b0 + bhh_b0)[None, :]
    tfa = _pack_stack(tf)
    tba = _pack_stack(tb)

    w1fa = wih_f1.T[:, :2 * H].astype(BF)
    w1fb = wih_f1.T[:, 2 * H:].astype(BF)
    w1ba = wih_b1.T[:, :2 * H].astype(BF)
    w1bb = wih_b1.T[:, 2 * H:].astype(BF)

    wf0p = _pack_stack(whh_f0.T)
    wb0p = _pack_stack(whh_b0.T)
    wf1p = _pack_stack(whh_f1.T)
    wb1p = _pack_stack(whh_b1.T)

    wpspec = pl.BlockSpec((2 * H, 2 * H), lambda i: (0, 0))
    idspec_f = pl.BlockSpec((1, CT, B), lambda i: (i, 0, 0))
    idspec_b = pl.BlockSpec((1, CT, B), lambda i: (NB - 1 - i, 0, 0))
    hspec_f = pl.BlockSpec((1, CT, B, H), lambda i: (i, 0, 0, 0))
    hspec_b = pl.BlockSpec((1, CT, B, H), lambda i: (NB - 1 - i, 0, 0, 0))

    hf, hb = pl.pallas_call(
        _l0_kernel,
        grid=(NB,),
        in_specs=[idspec_f, idspec_b,
                  wpspec, wpspec, wpspec, wpspec],
        out_specs=[hspec_f, hspec_b],
        out_shape=[jax.ShapeDtypeStruct((NB, CT, B, H), BF)] * 2,
        scratch_shapes=[pltpu.VMEM((2, B, H), F32),
                        pltpu.VMEM((2, B, H), BF)],
        compiler_params=pltpu.CompilerParams(
            dimension_semantics=("arbitrary",)),
    )(ids3, ids3, tfa, tba, wf0p, wb0p)

    bspec = pl.BlockSpec((1, G), lambda i: (0, 0))
    wlspec = pl.BlockSpec((1, H), lambda i: (0, 0))

    out3 = pl.pallas_call(
        _l1_kernel,
        grid=(NB,),
        in_specs=[hspec_f, hspec_f, hspec_b, hspec_b,
                  wpspec, wpspec, wpspec, wpspec, wpspec, wpspec,
                  bspec, bspec, wlspec, wlspec,
                  pl.BlockSpec((1, 1), lambda i: (0, 0))],
        out_specs=pl.BlockSpec((NB, CT, B), lambda i: (0, 0, 0)),
        out_shape=jax.ShapeDtypeStruct((NB, CT, B), F32),
        scratch_shapes=[
            pltpu.VMEM((CT, B, H), F32),
            pltpu.VMEM((CT, B, H), F32),
            pltpu.VMEM((2, B, H), F32),
            pltpu.VMEM((2, B, H), BF),
        ],
        compiler_params=pltpu.CompilerParams(
            dimension_semantics=("arbitrary",)),
    )(hf, hb, hf, hb,
      w1fa, w1fb, w1ba, w1bb,
      wf1p, wb1p,
      (bih_f1 + bhh_f1)[None, :], (bih_b1 + bhh_b1)[None, :],
      linear_W[:, :H], linear_W[:, H:], linear_b.reshape(1, 1))

    return out3.reshape(S, B).T
